# baseline (speedup 1.0000x reference)
import functools

import jax
import jax.numpy as jnp
from jax import lax
from jax.experimental import pallas as pl
from jax.experimental.pallas import tpu as pltpu
from jax.experimental.pallas import tpu_sc as plsc

N_USERS = 25000
N_ITEMS = 25000
N = N_USERS + N_ITEMS
DIM = 64
HID = 64
E = 800000
B = 16384

NC = 2
NS = 16
CHUNK = 128

E_PAD = 802816
EPT16 = E_PAD // 16
EPT32 = E_PAD // 32
NDEG = 50048
DEG_SLICE = NDEG // NS
SPAD = 50048
ROW_SLICE = SPAD // NS

_mesh = plsc.VectorSubcoreMesh(core_axis_name="c", subcore_axis_name="s")
_sc_params = pltpu.CompilerParams(use_tc_tiling_on_sc=False)
_sc_params_nl = pltpu.CompilerParams(use_tc_tiling_on_sc=False,
                                     needs_layout_passes=False)


DBLK = 28
DNPAIR = DBLK // 2
DNBLK = EPT32 // CHUNK // DBLK


def _deg_body(col_hbm, ew_hbm, out_hbm, colb, ewb, zbuf, deg_sh, ss0, ss1):
    c = lax.axis_index("c")
    s = lax.axis_index("s")
    wid = c * NS + s

    def _zero(i, _):
        zbuf[pl.ds(i * 16, 16)] = jnp.zeros((16,), jnp.float32)
        return _
    lax.fori_loop(0, DEG_SLICE // 16, _zero, None)
    pltpu.sync_copy(zbuf, deg_sh.at[pl.ds(s * DEG_SLICE, DEG_SLICE)])
    plsc.subcore_barrier()

    def _s_start(j, sem):
        pltpu.async_copy(ewb.at[j], deg_sh.at[colb.at[j]], sem, add=True)

    def _s_wait(sem):
        pltpu.make_async_copy(ewb.at[0], deg_sh.at[colb.at[0]], sem).wait()

    def _block(bi, _):
        cb = wid * (EPT32 // CHUNK) + bi * DBLK

        @pl.when(bi > 0)
        def _():
            _s_wait(ss0)
            _s_wait(ss1)

        pltpu.sync_copy(col_hbm.at[pl.ds(cb, DBLK)], colb)
        pltpu.sync_copy(ew_hbm.at[pl.ds(cb, DBLK)], ewb)

        def _pair(t, _):
            j0 = 2 * t

            @pl.when(t > 0)
            def _():
                _s_wait(ss0)
                _s_wait(ss1)
            _s_start(j0, ss0)
            _s_start(j0 + 1, ss1)
            return _
        lax.fori_loop(0, DNPAIR, _pair, None)
        return _
    lax.fori_loop(0, DNBLK, _block, None)

    _s_wait(ss0)
    _s_wait(ss1)

    plsc.subcore_barrier()
    pltpu.sync_copy(deg_sh.at[pl.ds(s * DEG_SLICE, DEG_SLICE)], zbuf)
    pltpu.sync_copy(zbuf, out_hbm.at[pl.ds(c * NDEG + s * DEG_SLICE, DEG_SLICE)])


def _sc_degree(col2, ew2):
    f = pl.kernel(
        _deg_body,
        compiler_params=_sc_params,
        out_type=jax.ShapeDtypeStruct((NC * NDEG,), jnp.float32),
        mesh=_mesh,
        scratch_types=[
            pltpu.VMEM((DBLK, CHUNK), jnp.int32),
            pltpu.VMEM((DBLK, CHUNK), jnp.float32),
            pltpu.VMEM((DEG_SLICE,), jnp.float32),
            pltpu.VMEM_SHARED((NDEG,), jnp.float32),
            pltpu.SemaphoreType.DMA,
            pltpu.SemaphoreType.DMA,
        ],
    )
    return f(col2, ew2)


BLK = 28
NQUAD = BLK // 4
CPT = EPT16 // CHUNK
NBLK = CPT // BLK

def _edge_body(row2_hbm, col2_hbm, ew2_hbm, xw_hbm, out_hbm,
               rowb, colb, ewb, gb0, gb1, gb2, gb3, y_sh,
               sg0, sg1, sg2, sg3, sc0, sc1, sc2, sc3):
    c = lax.axis_index("c")
    s = lax.axis_index("s")
    gbufs = (gb0, gb1, gb2, gb3)
    sgs = (sg0, sg1, sg2, sg3)
    sss = (sc0, sc1, sc2, sc3)

    pltpu.sync_copy(xw_hbm.at[pl.ds(c * SPAD + s * ROW_SLICE, ROW_SLICE)],
                    y_sh.at[pl.ds(s * ROW_SLICE, ROW_SLICE)])
    plsc.subcore_barrier()

    def _g_start(jrow, gbuf, sem):
        return pltpu.async_copy(xw_hbm.at[rowb.at[jrow]], gbuf, sem)

    def _g_wait(gbuf, sem):
        pltpu.make_async_copy(xw_hbm.at[rowb.at[0]], gbuf, sem).wait()

    def _s_start(gbuf, jcol, sem):
        return pltpu.async_copy(gbuf, y_sh.at[colb.at[jcol]], sem, add=True)

    def _s_wait(gbuf, sem):
        pltpu.make_async_copy(gbuf, y_sh.at[colb.at[0]], sem).wait()

    def _scale(gbuf, j):
        def grp(g, _):
            wv = ewb[j, pl.ds(g * 16, 16)]
            for l in range(16):
                i = g * 16 + l
                w = wv[l]
                gbuf[i, pl.ds(0, 16)] = gbuf[i, pl.ds(0, 16)] * w
                gbuf[i, pl.ds(16, 16)] = gbuf[i, pl.ds(16, 16)] * w
            return _
        for g in range(CHUNK // 16):
            grp(g, None)

    def _block(bi, _):
        rb = c * (E_PAD // CHUNK) + s * CPT + bi * BLK
        cb = s * CPT + bi * BLK

        @pl.when(bi > 0)
        def _():
            for q in range(4):
                _s_wait(gbufs[q], sss[q])

        pltpu.sync_copy(row2_hbm.at[pl.ds(rb, BLK)], rowb)
        pltpu.sync_copy(col2_hbm.at[pl.ds(cb, BLK)], colb)
        pltpu.sync_copy(ew2_hbm.at[pl.ds(cb, BLK)], ewb)

        _g_start(0, gbufs[0], sgs[0])
        _g_start(1, gbufs[1], sgs[1])
        _g_start(2, gbufs[2], sgs[2])

        def _quad(t, _):
            for q in range(4):
                j = 4 * t + q
                nb = (q + 3) % 4
                _g_wait(gbufs[q], sgs[q])

                def _prefetch():
                    _s_wait(gbufs[nb], sss[nb])
                    _g_start(j + 3, gbufs[nb], sgs[nb])

                if q == 0:
                    @pl.when(t > 0)
                    def _():
                        _s_wait(gbufs[nb], sss[nb])
                    _g_start(j + 3, gbufs[nb], sgs[nb])
                else:
                    @pl.when(t < NQUAD - 1)
                    def _():
                        _prefetch()
                _scale(gbufs[q], j)
                _s_start(gbufs[q], j, sss[q])
            return _
        lax.fori_loop(0, NQUAD, _quad, None)
        return _
    lax.fori_loop(0, NBLK, _block, None)

    for q in range(4):
        _s_wait(gbufs[q], sss[q])

    plsc.subcore_barrier()
    pltpu.sync_copy(y_sh.at[pl.ds(s * ROW_SLICE, ROW_SLICE)],
                    out_hbm.at[pl.ds(c * SPAD + s * ROW_SLICE, ROW_SLICE)])


def _sc_edge(rowcat2, col2, ew2, xw_flat):
    f = pl.kernel(
        _edge_body,
        compiler_params=_sc_params_nl,
        out_type=jax.ShapeDtypeStruct((NC * SPAD, 32), jnp.float32),
        mesh=_mesh,
        scratch_types=[
            pltpu.VMEM((BLK, CHUNK), jnp.int32),
            pltpu.VMEM((BLK, CHUNK), jnp.int32),
            pltpu.VMEM((BLK, CHUNK), jnp.float32),
            pltpu.VMEM((CHUNK, 32), jnp.float32),
            pltpu.VMEM((CHUNK, 32), jnp.float32),
            pltpu.VMEM((CHUNK, 32), jnp.float32),
            pltpu.VMEM((CHUNK, 32), jnp.float32),
            pltpu.VMEM_SHARED((SPAD, 32), jnp.float32),
        ] + [pltpu.SemaphoreType.DMA] * 8,
    )
    return f(rowcat2, col2, ew2, xw_flat)


BN = 2000


def _xw_body(x_ref, w_ref, degt_ref, out_ref):
    d = degt_ref[...]
    dinv = lax.rsqrt(jnp.sum(d, axis=1) + 1.0)
    xw = jnp.dot(x_ref[...], w_ref[...], preferred_element_type=jnp.float32)
    xw = xw * dinv[:, None]
    out_ref[0] = xw[:, :32]
    out_ref[1] = xw[:, 32:]


def _tc_xw(x, W, degt):
    return pl.pallas_call(
        _xw_body,
        out_shape=jax.ShapeDtypeStruct((NC, SPAD, 32), jnp.float32),
        grid=(N // BN,),
        in_specs=[
            pl.BlockSpec((BN, DIM), lambda j: (j, 0)),
            pl.BlockSpec((DIM, HID), lambda j: (0, 0)),
            pl.BlockSpec((BN, NC), lambda j: (j, 0)),
        ],
        out_specs=pl.BlockSpec((NC, BN, 32), lambda j: (0, j, 0)),
    )(x, W, degt)


def _post_body(y_ref, degt_ref, b_ref, g_ref, be_ref, hin_ref, out_ref):
    d = degt_ref[...]
    dinv = lax.rsqrt(jnp.sum(d, axis=1) + 1.0)
    y = jnp.concatenate([y_ref[0], y_ref[1]], axis=1)
    pre = y * dinv[:, None] + b_ref[...][None, :]
    m = jnp.mean(pre, axis=1, keepdims=True)
    v = jnp.mean((pre - m) ** 2, axis=1, keepdims=True)
    h = (pre - m) * lax.rsqrt(v + 1e-5) * g_ref[...][None, :] + be_ref[...][None, :]
    out_ref[...] = jnp.maximum(h, 0.0) + hin_ref[...]


def _post_xw_body(y_ref, degt_ref, b_ref, g_ref, be_ref, hin_ref, w_ref,
                  h_ref, out_ref):
    d = degt_ref[...]
    dinv = lax.rsqrt(jnp.sum(d, axis=1) + 1.0)
    y = jnp.concatenate([y_ref[0], y_ref[1]], axis=1)
    pre = y * dinv[:, None] + b_ref[...][None, :]
    m = jnp.mean(pre, axis=1, keepdims=True)
    v = jnp.mean((pre - m) ** 2, axis=1, keepdims=True)
    h = (pre - m) * lax.rsqrt(v + 1e-5) * g_ref[...][None, :] + be_ref[...][None, :]
    h = jnp.maximum(h, 0.0) + hin_ref[...]
    h_ref[...] = h
    xw = jnp.dot(h, w_ref[...], preferred_element_type=jnp.float32)
    xw = xw * dinv[:, None]
    out_ref[0] = xw[:, :32]
    out_ref[1] = xw[:, 32:]


def _tc_post_xw(y2, degt, b, g, be, hin, W):
    return pl.pallas_call(
        _post_xw_body,
        out_shape=(jax.ShapeDtypeStruct((N, HID), jnp.float32),
                   jax.ShapeDtypeStruct((NC, SPAD, 32), jnp.float32)),
        grid=(SPAD // BN,),
        in_specs=[
            pl.BlockSpec((NC, BN, 32), lambda j: (0, j, 0)),
            pl.BlockSpec((BN, NC), lambda j: (j, 0)),
            pl.BlockSpec((HID,), lambda j: (0,)),
            pl.BlockSpec((HID,), lambda j: (0,)),
            pl.BlockSpec((HID,), lambda j: (0,)),
            pl.BlockSpec((BN, HID), lambda j: (j, 0)),
            pl.BlockSpec((HID, HID), lambda j: (0, 0)),
        ],
        out_specs=(pl.BlockSpec((BN, HID), lambda j: (j, 0)),
                   pl.BlockSpec((NC, BN, 32), lambda j: (0, j, 0))),
    )(y2, degt, b, g, be, hin, W)


def _post_proj_body(y_ref, degt_ref, b_ref, g_ref, be_ref, hin_ref,
                    wp_ref, bp_ref, out_ref):
    d = degt_ref[...]
    dinv = lax.rsqrt(jnp.sum(d, axis=1) + 1.0)
    y = jnp.concatenate([y_ref[0], y_ref[1]], axis=1)
    pre = y * dinv[:, None] + b_ref[...][None, :]
    m = jnp.mean(pre, axis=1, keepdims=True)
    v = jnp.mean((pre - m) ** 2, axis=1, keepdims=True)
    h = (pre - m) * lax.rsqrt(v + 1e-5) * g_ref[...][None, :] + be_ref[...][None, :]
    h = jnp.maximum(h, 0.0) + hin_ref[...]
    p = jnp.dot(h, wp_ref[...], preferred_element_type=jnp.float32)
    out_ref[...] = p + bp_ref[...][None, :]


def _tc_post_proj(y2, degt, b, g, be, hin, Wp, bp):
    return pl.pallas_call(
        _post_proj_body,
        out_shape=jax.ShapeDtypeStruct((N, DIM), jnp.float32),
        grid=(SPAD // BN,),
        in_specs=[
            pl.BlockSpec((NC, BN, 32), lambda j: (0, j, 0)),
            pl.BlockSpec((BN, NC), lambda j: (j, 0)),
            pl.BlockSpec((HID,), lambda j: (0,)),
            pl.BlockSpec((HID,), lambda j: (0,)),
            pl.BlockSpec((HID,), lambda j: (0,)),
            pl.BlockSpec((BN, HID), lambda j: (j, 0)),
            pl.BlockSpec((HID, DIM), lambda j: (0, 0)),
            pl.BlockSpec((DIM,), lambda j: (0,)),
        ],
        out_specs=pl.BlockSpec((BN, DIM), lambda j: (j, 0)),
    )(y2, degt, b, g, be, hin, Wp, bp)


PPT = B // (NC * NS)
PCHUNKS = PPT // CHUNK


def _pred_body(u_hbm, i_hbm, ia_hbm, p_hbm, bu_hbm, bi_hbm, mu_hbm, out_hbm,
               uv0, uv1, iv0, iv1, ia0, ia1, pub0, pub1, pib0, pib1,
               buv, biv, muv, sbuf, sm0, sm1):
    c = lax.axis_index("c")
    s = lax.axis_index("s")
    wid = c * NS + s
    uvs, ivs, ias = (uv0, uv1), (iv0, iv1), (ia0, ia1)
    pubs, pibs = (pub0, pub1), (pib0, pib1)
    sems = (sm0, sm1)

    pltpu.sync_copy(bu_hbm, buv)
    pltpu.sync_copy(bi_hbm, biv)
    pltpu.sync_copy(mu_hbm, muv)
    mu_vec = muv[...]

    def _stage(kk, b):
        base = wid * PPT + kk * CHUNK
        pltpu.sync_copy(u_hbm.at[pl.ds(base, CHUNK)], uvs[b])
        pltpu.sync_copy(i_hbm.at[pl.ds(base, CHUNK)], ivs[b])
        pltpu.sync_copy(ia_hbm.at[pl.ds(base, CHUNK)], ias[b])
        pltpu.async_copy(p_hbm.at[uvs[b]], pubs[b], sems[b])
        pltpu.async_copy(p_hbm.at[ias[b]], pibs[b], sems[b])

    def _dot_chunk(kk, b):
        pltpu.make_async_copy(p_hbm.at[uvs[b]], pubs[b], sems[b]).wait()
        pltpu.make_async_copy(p_hbm.at[ias[b]], pibs[b], sems[b]).wait()

        def _dot(j, _):
            rows = lax.iota(jnp.int32, 16) + j * 16
            acc = jnp.zeros((16,), jnp.float32)
            for f in range(DIM):
                fv = jnp.full((16,), f, jnp.int32)
                a = plsc.load_gather(pubs[b], [rows, fv])
                bvals = plsc.load_gather(pibs[b], [rows, fv])
                acc = acc + a * bvals
            uv16 = uvs[b][pl.ds(j * 16, 16)]
            iv16 = ivs[b][pl.ds(j * 16, 16)]
            bu16 = plsc.load_gather(buv, [uv16])
            bi16 = plsc.load_gather(biv, [iv16])
            sc = acc + bu16 + bi16 + mu_vec
            sc = jnp.minimum(jnp.maximum(sc, 1.0), 5.0)
            sbuf[pl.ds(kk * CHUNK + j * 16, 16)] = sc
            return _
        lax.fori_loop(0, CHUNK // 16, _dot, None)

    _stage(0, 0)
    for kk in range(PCHUNKS):
        if kk + 1 < PCHUNKS:
            _stage(kk + 1, (kk + 1) % 2)
        _dot_chunk(kk, kk % 2)

    pltpu.sync_copy(sbuf, out_hbm.at[pl.ds(wid * PPT, PPT)])


def _sc_predict(users, items, items_adj, p, bu_flat, bi_flat, mu16):
    f = pl.kernel(
        _pred_body,
        compiler_params=_sc_params_nl,
        out_type=jax.ShapeDtypeStruct((B,), jnp.float32),
        mesh=_mesh,
        scratch_types=[
            pltpu.VMEM((CHUNK,), jnp.int32),
            pltpu.VMEM((CHUNK,), jnp.int32),
            pltpu.VMEM((CHUNK,), jnp.int32),
            pltpu.VMEM((CHUNK,), jnp.int32),
            pltpu.VMEM((CHUNK,), jnp.int32),
            pltpu.VMEM((CHUNK,), jnp.int32),
            pltpu.VMEM((CHUNK, DIM), jnp.float32),
            pltpu.VMEM((CHUNK, DIM), jnp.float32),
            pltpu.VMEM((CHUNK, DIM), jnp.float32),
            pltpu.VMEM((CHUNK, DIM), jnp.float32),
            pltpu.VMEM((N_USERS,), jnp.float32),
            pltpu.VMEM((N_ITEMS,), jnp.float32),
            pltpu.VMEM((16,), jnp.float32),
            pltpu.VMEM((PPT,), jnp.float32),
            pltpu.SemaphoreType.DMA,
            pltpu.SemaphoreType.DMA,
        ],
    )
    return f(users, items, items_adj, p, bu_flat, bi_flat, mu16)


def kernel(users, items, edge_index, edge_weight, user_emb, item_emb,
           W1, b1, g1, be1, W2, b2, g2, be2, Wp, bp, bu, bi, mu):
    row = edge_index[0].astype(jnp.int32)
    col = edge_index[1].astype(jnp.int32)
    pad = E_PAD - E
    row_p = jnp.pad(row, (0, pad))
    col_p = jnp.pad(col, (0, pad))
    ew_p = jnp.pad(edge_weight, (0, pad))
    rowcat2 = jnp.reshape(
        jnp.concatenate([row_p, row_p + SPAD]), (2 * E_PAD // CHUNK, CHUNK))
    col2 = jnp.reshape(col_p, (E_PAD // CHUNK, CHUNK))
    ew2 = jnp.reshape(ew_p, (E_PAD // CHUNK, CHUNK))

    x = jnp.concatenate([user_emb, item_emb], axis=0)

    degpart = _sc_degree(col2, ew2)
    degt = jnp.reshape(degpart, (NC, NDEG))[:, :N].T

    xwh1 = _tc_xw(x, W1, degt)
    y1 = _sc_edge(rowcat2, col2, ew2, jnp.reshape(xwh1, (NC * SPAD, 32)))
    h1, xwh2 = _tc_post_xw(jnp.reshape(y1, (NC, SPAD, 32)), degt,
                           b1, g1, be1, x, W2)
    y2 = _sc_edge(rowcat2, col2, ew2, jnp.reshape(xwh2, (NC * SPAD, 32)))
    p = _tc_post_proj(jnp.reshape(y2, (NC, SPAD, 32)), degt,
                      b2, g2, be2, h1, Wp, bp)

    mu16 = jnp.broadcast_to(mu, (16,)).astype(jnp.float32)
    users_i = users.astype(jnp.int32)
    items_i = items.astype(jnp.int32)
    s = _sc_predict(users_i, items_i, items_i + N_USERS, p,
                    bu[:, 0], bi[:, 0], mu16)
    return s


# speedup vs baseline: 1.0084x; 1.0084x over previous
import functools

import jax
import jax.numpy as jnp
from jax import lax
from jax.experimental import pallas as pl
from jax.experimental.pallas import tpu as pltpu
from jax.experimental.pallas import tpu_sc as plsc

N_USERS = 25000
N_ITEMS = 25000
N = N_USERS + N_ITEMS
DIM = 64
HID = 64
E = 800000
B = 16384

NC = 2
NS = 16
CHUNK = 128

E_PAD = 802816
EPT16 = E_PAD // 16
EPT32 = E_PAD // 32
NDEG = 50048
DEG_SLICE = NDEG // NS
SPAD = 50048
ROW_SLICE = SPAD // NS

_mesh = plsc.VectorSubcoreMesh(core_axis_name="c", subcore_axis_name="s")
_sc_params = pltpu.CompilerParams(use_tc_tiling_on_sc=False)
_sc_params_nl = pltpu.CompilerParams(use_tc_tiling_on_sc=False,
                                     needs_layout_passes=False)


DBLK = 28
DNPAIR = DBLK // 2
DNBLK = EPT32 // CHUNK // DBLK


def _deg_body(col_hbm, ew_hbm, out_hbm, colb, ewb, zbuf, deg_sh, ss0, ss1):
    c = lax.axis_index("c")
    s = lax.axis_index("s")
    wid = c * NS + s

    def _zero(i, _):
        zbuf[pl.ds(i * 16, 16)] = jnp.zeros((16,), jnp.float32)
        return _
    lax.fori_loop(0, DEG_SLICE // 16, _zero, None)
    pltpu.sync_copy(zbuf, deg_sh.at[pl.ds(s * DEG_SLICE, DEG_SLICE)])
    plsc.subcore_barrier()

    def _s_start(j, sem):
        pltpu.async_copy(ewb.at[j], deg_sh.at[colb.at[j]], sem, add=True)

    def _s_wait(sem):
        pltpu.make_async_copy(ewb.at[0], deg_sh.at[colb.at[0]], sem).wait()

    def _block(bi, _):
        cb = wid * (EPT32 // CHUNK) + bi * DBLK

        @pl.when(bi > 0)
        def _():
            _s_wait(ss0)
            _s_wait(ss1)

        pltpu.sync_copy(col_hbm.at[pl.ds(cb, DBLK)], colb)
        pltpu.sync_copy(ew_hbm.at[pl.ds(cb, DBLK)], ewb)

        def _pair(t, _):
            j0 = 2 * t

            @pl.when(t > 0)
            def _():
                _s_wait(ss0)
                _s_wait(ss1)
            _s_start(j0, ss0)
            _s_start(j0 + 1, ss1)
            return _
        lax.fori_loop(0, DNPAIR, _pair, None)
        return _
    lax.fori_loop(0, DNBLK, _block, None)

    _s_wait(ss0)
    _s_wait(ss1)

    plsc.subcore_barrier()
    pltpu.sync_copy(deg_sh.at[pl.ds(s * DEG_SLICE, DEG_SLICE)], zbuf)
    pltpu.sync_copy(zbuf, out_hbm.at[pl.ds(c * NDEG + s * DEG_SLICE, DEG_SLICE)])


def _sc_degree(col2, ew2):
    f = pl.kernel(
        _deg_body,
        compiler_params=_sc_params,
        out_type=jax.ShapeDtypeStruct((NC * NDEG,), jnp.float32),
        mesh=_mesh,
        scratch_types=[
            pltpu.VMEM((DBLK, CHUNK), jnp.int32),
            pltpu.VMEM((DBLK, CHUNK), jnp.float32),
            pltpu.VMEM((DEG_SLICE,), jnp.float32),
            pltpu.VMEM_SHARED((NDEG,), jnp.float32),
            pltpu.SemaphoreType.DMA,
            pltpu.SemaphoreType.DMA,
        ],
    )
    return f(col2, ew2)


BLK = 28
NQUAD = BLK // 4
CPT = EPT16 // CHUNK
NBLK = CPT // BLK

def _edge_body(row2_hbm, col2_hbm, ew2_hbm, xw_hbm, out_hbm,
               rowb, colb, ewb, gb0, gb1, gb2, gb3, y_sh,
               sg0, sg1, sg2, sg3, sc0, sc1, sc2, sc3):
    c = lax.axis_index("c")
    s = lax.axis_index("s")
    gbufs = (gb0, gb1, gb2, gb3)
    sgs = (sg0, sg1, sg2, sg3)
    sss = (sc0, sc1, sc2, sc3)

    pltpu.sync_copy(xw_hbm.at[pl.ds(c * SPAD + s * ROW_SLICE, ROW_SLICE)],
                    y_sh.at[pl.ds(s * ROW_SLICE, ROW_SLICE)])
    plsc.subcore_barrier()

    def _g_start(jrow, gbuf, sem):
        return pltpu.async_copy(xw_hbm.at[rowb.at[jrow]], gbuf, sem)

    def _g_wait(gbuf, sem):
        pltpu.make_async_copy(xw_hbm.at[rowb.at[0]], gbuf, sem).wait()

    def _s_start(gbuf, jcol, sem):
        return pltpu.async_copy(gbuf, y_sh.at[colb.at[jcol]], sem, add=True)

    def _s_wait(gbuf, sem):
        pltpu.make_async_copy(gbuf, y_sh.at[colb.at[0]], sem).wait()

    def _scale(gbuf, j):
        def grp(g, _):
            wv = ewb[j, pl.ds(g * 16, 16)]
            for l in range(16):
                i = g * 16 + l
                w = wv[l]
                gbuf[i, pl.ds(0, 16)] = gbuf[i, pl.ds(0, 16)] * w
                gbuf[i, pl.ds(16, 16)] = gbuf[i, pl.ds(16, 16)] * w
            return _
        for g in range(CHUNK // 16):
            grp(g, None)

    def _block(bi, _):
        rb = c * (E_PAD // CHUNK) + s * CPT + bi * BLK
        cb = s * CPT + bi * BLK

        @pl.when(bi > 0)
        def _():
            for q in range(4):
                _s_wait(gbufs[q], sss[q])

        pltpu.sync_copy(row2_hbm.at[pl.ds(rb, BLK)], rowb)
        pltpu.sync_copy(col2_hbm.at[pl.ds(cb, BLK)], colb)
        pltpu.sync_copy(ew2_hbm.at[pl.ds(cb, BLK)], ewb)

        _g_start(0, gbufs[0], sgs[0])
        _g_start(1, gbufs[1], sgs[1])
        _g_start(2, gbufs[2], sgs[2])

        def _quad(t, _):
            for q in range(4):
                j = 4 * t + q
                nb = (q + 3) % 4
                _g_wait(gbufs[q], sgs[q])

                def _prefetch():
                    _s_wait(gbufs[nb], sss[nb])
                    _g_start(j + 3, gbufs[nb], sgs[nb])

                if q == 0:
                    @pl.when(t > 0)
                    def _():
                        _s_wait(gbufs[nb], sss[nb])
                    _g_start(j + 3, gbufs[nb], sgs[nb])
                else:
                    @pl.when(t < NQUAD - 1)
                    def _():
                        _prefetch()
                _scale(gbufs[q], j)
                _s_start(gbufs[q], j, sss[q])
            return _
        lax.fori_loop(0, NQUAD, _quad, None)
        return _
    lax.fori_loop(0, NBLK, _block, None)

    for q in range(4):
        _s_wait(gbufs[q], sss[q])

    plsc.subcore_barrier()
    pltpu.sync_copy(y_sh.at[pl.ds(s * ROW_SLICE, ROW_SLICE)],
                    out_hbm.at[pl.ds(c * SPAD + s * ROW_SLICE, ROW_SLICE)])


def _sc_edge(rowcat2, col2, ew2, xw_flat):
    f = pl.kernel(
        _edge_body,
        compiler_params=_sc_params_nl,
        out_type=jax.ShapeDtypeStruct((NC * SPAD, 32), jnp.float32),
        mesh=_mesh,
        scratch_types=[
            pltpu.VMEM((BLK, CHUNK), jnp.int32),
            pltpu.VMEM((BLK, CHUNK), jnp.int32),
            pltpu.VMEM((BLK, CHUNK), jnp.float32),
            pltpu.VMEM((CHUNK, 32), jnp.float32),
            pltpu.VMEM((CHUNK, 32), jnp.float32),
            pltpu.VMEM((CHUNK, 32), jnp.float32),
            pltpu.VMEM((CHUNK, 32), jnp.float32),
            pltpu.VMEM_SHARED((SPAD, 32), jnp.float32),
        ] + [pltpu.SemaphoreType.DMA] * 8,
    )
    return f(rowcat2, col2, ew2, xw_flat)


BN = 2000


def _xw_body(x_ref, w_ref, degt_ref, out_ref):
    d = degt_ref[...]
    dinv = lax.rsqrt(jnp.sum(d, axis=1) + 1.0)
    xw = jnp.dot(x_ref[...], w_ref[...], preferred_element_type=jnp.float32)
    xw = xw * dinv[:, None]
    out_ref[0] = xw[:, :32]
    out_ref[1] = xw[:, 32:]


def _tc_xw(x, W, degt):
    return pl.pallas_call(
        _xw_body,
        out_shape=jax.ShapeDtypeStruct((NC, SPAD, 32), jnp.float32),
        grid=(N // BN,),
        in_specs=[
            pl.BlockSpec((BN, DIM), lambda j: (j, 0)),
            pl.BlockSpec((DIM, HID), lambda j: (0, 0)),
            pl.BlockSpec((BN, NC), lambda j: (j, 0)),
        ],
        out_specs=pl.BlockSpec((NC, BN, 32), lambda j: (0, j, 0)),
    )(x, W, degt)


def _post_body(y_ref, degt_ref, b_ref, g_ref, be_ref, hin_ref, out_ref):
    d = degt_ref[...]
    dinv = lax.rsqrt(jnp.sum(d, axis=1) + 1.0)
    y = jnp.concatenate([y_ref[0], y_ref[1]], axis=1)
    pre = y * dinv[:, None] + b_ref[...][None, :]
    m = jnp.mean(pre, axis=1, keepdims=True)
    v = jnp.mean((pre - m) ** 2, axis=1, keepdims=True)
    h = (pre - m) * lax.rsqrt(v + 1e-5) * g_ref[...][None, :] + be_ref[...][None, :]
    out_ref[...] = jnp.maximum(h, 0.0) + hin_ref[...]


def _post_xw_body(y_ref, degt_ref, b_ref, g_ref, be_ref, hin_ref, w_ref,
                  h_ref, out_ref):
    d = degt_ref[...]
    dinv = lax.rsqrt(jnp.sum(d, axis=1) + 1.0)
    y = jnp.concatenate([y_ref[0], y_ref[1]], axis=1)
    pre = y * dinv[:, None] + b_ref[...][None, :]
    m = jnp.mean(pre, axis=1, keepdims=True)
    v = jnp.mean((pre - m) ** 2, axis=1, keepdims=True)
    h = (pre - m) * lax.rsqrt(v + 1e-5) * g_ref[...][None, :] + be_ref[...][None, :]
    h = jnp.maximum(h, 0.0) + hin_ref[...]
    h_ref[...] = h
    xw = jnp.dot(h, w_ref[...], preferred_element_type=jnp.float32)
    xw = xw * dinv[:, None]
    out_ref[0] = xw[:, :32]
    out_ref[1] = xw[:, 32:]


def _tc_post_xw(y2, degt, b, g, be, hin, W):
    return pl.pallas_call(
        _post_xw_body,
        out_shape=(jax.ShapeDtypeStruct((N, HID), jnp.float32),
                   jax.ShapeDtypeStruct((NC, SPAD, 32), jnp.float32)),
        grid=(SPAD // BN,),
        in_specs=[
            pl.BlockSpec((NC, BN, 32), lambda j: (0, j, 0)),
            pl.BlockSpec((BN, NC), lambda j: (j, 0)),
            pl.BlockSpec((HID,), lambda j: (0,)),
            pl.BlockSpec((HID,), lambda j: (0,)),
            pl.BlockSpec((HID,), lambda j: (0,)),
            pl.BlockSpec((BN, HID), lambda j: (j, 0)),
            pl.BlockSpec((HID, HID), lambda j: (0, 0)),
        ],
        out_specs=(pl.BlockSpec((BN, HID), lambda j: (j, 0)),
                   pl.BlockSpec((NC, BN, 32), lambda j: (0, j, 0))),
    )(y2, degt, b, g, be, hin, W)


def _post_proj_body(y_ref, degt_ref, b_ref, g_ref, be_ref, hin_ref,
                    wp_ref, bp_ref, out_ref):
    d = degt_ref[...]
    dinv = lax.rsqrt(jnp.sum(d, axis=1) + 1.0)
    y = jnp.concatenate([y_ref[0], y_ref[1]], axis=1)
    pre = y * dinv[:, None] + b_ref[...][None, :]
    m = jnp.mean(pre, axis=1, keepdims=True)
    v = jnp.mean((pre - m) ** 2, axis=1, keepdims=True)
    h = (pre - m) * lax.rsqrt(v + 1e-5) * g_ref[...][None, :] + be_ref[...][None, :]
    h = jnp.maximum(h, 0.0) + hin_ref[...]
    p = jnp.dot(h, wp_ref[...], preferred_element_type=jnp.float32)
    out_ref[...] = p + bp_ref[...][None, :]


def _tc_post_proj(y2, degt, b, g, be, hin, Wp, bp):
    return pl.pallas_call(
        _post_proj_body,
        out_shape=jax.ShapeDtypeStruct((N, DIM), jnp.float32),
        grid=(SPAD // BN,),
        in_specs=[
            pl.BlockSpec((NC, BN, 32), lambda j: (0, j, 0)),
            pl.BlockSpec((BN, NC), lambda j: (j, 0)),
            pl.BlockSpec((HID,), lambda j: (0,)),
            pl.BlockSpec((HID,), lambda j: (0,)),
            pl.BlockSpec((HID,), lambda j: (0,)),
            pl.BlockSpec((BN, HID), lambda j: (j, 0)),
            pl.BlockSpec((HID, DIM), lambda j: (0, 0)),
            pl.BlockSpec((DIM,), lambda j: (0,)),
        ],
        out_specs=pl.BlockSpec((BN, DIM), lambda j: (j, 0)),
    )(y2, degt, b, g, be, hin, Wp, bp)


PPT = B // (NC * NS)
PCHUNKS = PPT // CHUNK


def _pred_body(u_hbm, i_hbm, ia_hbm, p_hbm, bu_hbm, bi_hbm, mu_hbm, out_hbm,
               uv0, uv1, iv0, iv1, ia0, ia1, pub0, pub1, pib0, pib1,
               bub0, bub1, bib0, bib1, muv, sbuf, sm0, sm1):
    c = lax.axis_index("c")
    s = lax.axis_index("s")
    wid = c * NS + s
    uvs, ivs, ias = (uv0, uv1), (iv0, iv1), (ia0, ia1)
    pubs, pibs = (pub0, pub1), (pib0, pib1)
    bubs, bibs = (bub0, bub1), (bib0, bib1)
    sems = (sm0, sm1)

    pltpu.sync_copy(mu_hbm, muv)
    mu_vec = muv[...]

    def _stage(kk, b):
        base = wid * PPT + kk * CHUNK
        pltpu.sync_copy(u_hbm.at[pl.ds(base, CHUNK)], uvs[b])
        pltpu.sync_copy(i_hbm.at[pl.ds(base, CHUNK)], ivs[b])
        pltpu.sync_copy(ia_hbm.at[pl.ds(base, CHUNK)], ias[b])
        pltpu.async_copy(p_hbm.at[uvs[b]], pubs[b], sems[b])
        pltpu.async_copy(p_hbm.at[ias[b]], pibs[b], sems[b])
        pltpu.async_copy(bu_hbm.at[uvs[b]], bubs[b], sems[b])
        pltpu.async_copy(bi_hbm.at[ivs[b]], bibs[b], sems[b])

    def _dot_chunk(kk, b):
        pltpu.make_async_copy(p_hbm.at[uvs[b]], pubs[b], sems[b]).wait()
        pltpu.make_async_copy(p_hbm.at[ias[b]], pibs[b], sems[b]).wait()
        pltpu.make_async_copy(bu_hbm.at[uvs[b]], bubs[b], sems[b]).wait()
        pltpu.make_async_copy(bi_hbm.at[ivs[b]], bibs[b], sems[b]).wait()

        def _dot(j, _):
            rows = lax.iota(jnp.int32, 16) + j * 16
            acc = jnp.zeros((16,), jnp.float32)
            for f in range(DIM):
                fv = jnp.full((16,), f, jnp.int32)
                a = plsc.load_gather(pubs[b], [rows, fv])
                bvals = plsc.load_gather(pibs[b], [rows, fv])
                acc = acc + a * bvals
            bu16 = bubs[b][pl.ds(j * 16, 16)]
            bi16 = bibs[b][pl.ds(j * 16, 16)]
            sc = acc + bu16 + bi16 + mu_vec
            sc = jnp.minimum(jnp.maximum(sc, 1.0), 5.0)
            sbuf[pl.ds(kk * CHUNK + j * 16, 16)] = sc
            return _
        lax.fori_loop(0, CHUNK // 16, _dot, None)

    _stage(0, 0)
    for kk in range(PCHUNKS):
        if kk + 1 < PCHUNKS:
            _stage(kk + 1, (kk + 1) % 2)
        _dot_chunk(kk, kk % 2)

    pltpu.sync_copy(sbuf, out_hbm.at[pl.ds(wid * PPT, PPT)])


def _sc_predict(users, items, items_adj, p, bu_flat, bi_flat, mu16):
    f = pl.kernel(
        _pred_body,
        compiler_params=_sc_params_nl,
        out_type=jax.ShapeDtypeStruct((B,), jnp.float32),
        mesh=_mesh,
        scratch_types=[
            pltpu.VMEM((CHUNK,), jnp.int32),
            pltpu.VMEM((CHUNK,), jnp.int32),
            pltpu.VMEM((CHUNK,), jnp.int32),
            pltpu.VMEM((CHUNK,), jnp.int32),
            pltpu.VMEM((CHUNK,), jnp.int32),
            pltpu.VMEM((CHUNK,), jnp.int32),
            pltpu.VMEM((CHUNK, DIM), jnp.float32),
            pltpu.VMEM((CHUNK, DIM), jnp.float32),
            pltpu.VMEM((CHUNK, DIM), jnp.float32),
            pltpu.VMEM((CHUNK, DIM), jnp.float32),
            pltpu.VMEM((CHUNK,), jnp.float32),
            pltpu.VMEM((CHUNK,), jnp.float32),
            pltpu.VMEM((CHUNK,), jnp.float32),
            pltpu.VMEM((CHUNK,), jnp.float32),
            pltpu.VMEM((16,), jnp.float32),
            pltpu.VMEM((PPT,), jnp.float32),
            pltpu.SemaphoreType.DMA,
            pltpu.SemaphoreType.DMA,
        ],
    )
    return f(users, items, items_adj, p, bu_flat, bi_flat, mu16)


def kernel(users, items, edge_index, edge_weight, user_emb, item_emb,
           W1, b1, g1, be1, W2, b2, g2, be2, Wp, bp, bu, bi, mu):
    row = edge_index[0].astype(jnp.int32)
    col = edge_index[1].astype(jnp.int32)
    pad = E_PAD - E
    row_p = jnp.pad(row, (0, pad))
    col_p = jnp.pad(col, (0, pad))
    ew_p = jnp.pad(edge_weight, (0, pad))
    rowcat2 = jnp.reshape(
        jnp.concatenate([row_p, row_p + SPAD]), (2 * E_PAD // CHUNK, CHUNK))
    col2 = jnp.reshape(col_p, (E_PAD // CHUNK, CHUNK))
    ew2 = jnp.reshape(ew_p, (E_PAD // CHUNK, CHUNK))

    x = jnp.concatenate([user_emb, item_emb], axis=0)

    degpart = _sc_degree(col2, ew2)
    degt = jnp.reshape(degpart, (NC, NDEG))[:, :N].T

    xwh1 = _tc_xw(x, W1, degt)
    y1 = _sc_edge(rowcat2, col2, ew2, jnp.reshape(xwh1, (NC * SPAD, 32)))
    h1, xwh2 = _tc_post_xw(jnp.reshape(y1, (NC, SPAD, 32)), degt,
                           b1, g1, be1, x, W2)
    y2 = _sc_edge(rowcat2, col2, ew2, jnp.reshape(xwh2, (NC * SPAD, 32)))
    p = _tc_post_proj(jnp.reshape(y2, (NC, SPAD, 32)), degt,
                      b2, g2, be2, h1, Wp, bp)

    mu16 = jnp.broadcast_to(mu, (16,)).astype(jnp.float32)
    users_i = users.astype(jnp.int32)
    items_i = items.astype(jnp.int32)
    s = _sc_predict(users_i, items_i, items_i + N_USERS, p,
                    bu[:, 0], bi[:, 0], mu16)
    return s
